# Initial kernel scaffold; baseline (speedup 1.0000x reference)
import jax
import jax.numpy as jnp
from jax.experimental import pallas as pl


def kernel(x, lengths, promptList, batchSize):
    raise NotImplementedError("write your pallas kernel here")


# speedup vs baseline: 3.5772x; 3.5772x over previous
import jax
import jax.numpy as jnp
from jax import lax
from jax.experimental import pallas as pl
from jax.experimental.pallas import tpu as pltpu, tpu_sc as plsc

_N = 50000
_L = 20
_D = 50
_BS = 500
_NB = _N // _BS
_C = 16
_NCHUNK = _N // _C
_NC = 2
_NS = 16
_NW = _NC * _NS


def _gather_body(x3, idx, lengths, outx, outlen, idx_v, rows_v, len_v,
                 sem_r, sem_l):
    wid = lax.axis_index("s") * _NC + lax.axis_index("c")
    ntrips = (_NCHUNK - wid + _NW - 1) // _NW

    def body(i, carry):
        base = (wid + i * _NW) * _C
        pltpu.sync_copy(idx.at[pl.ds(base, _C)], idx_v)
        cp_r = pltpu.async_copy(x3.at[idx_v], rows_v, sem_r)
        cp_l = pltpu.async_copy(lengths.at[idx_v], len_v, sem_l)
        cp_r.wait()
        cp_l.wait()
        pltpu.sync_copy(rows_v, outx.at[pl.ds(base, _C)])
        pltpu.sync_copy(len_v, outlen.at[pl.ds(base, _C)])
        return carry

    lax.fori_loop(0, ntrips, body, 0)


def kernel(x, lengths, promptList, batchSize):
    x = x.reshape(_N, _L * _D)
    idx = promptList.astype(jnp.int32)
    mesh = plsc.VectorSubcoreMesh(core_axis_name="c", subcore_axis_name="s")
    outx, outlen = pl.kernel(
        _gather_body,
        out_type=(
            jax.ShapeDtypeStruct((_N, _L * _D), jnp.float32),
            jax.ShapeDtypeStruct((_N,), jnp.int32),
        ),
        mesh=mesh,
        scratch_types=[
            pltpu.VMEM((_C,), jnp.int32),
            pltpu.VMEM((_C, _L * _D), jnp.float32),
            pltpu.VMEM((_C,), jnp.int32),
            pltpu.SemaphoreType.DMA,
            pltpu.SemaphoreType.DMA,
        ],
        compiler_params=pltpu.CompilerParams(use_tc_tiling_on_sc=False),
    )(x, idx, lengths)
    xList = outx.reshape(_NB, _BS, _L, _D)
    lengthList = outlen.reshape(_NB, _BS) + jnp.asarray(
        batchSize - _BS, dtype=outlen.dtype)
    return (xList, lengthList)


# speedup vs baseline: 4.9813x; 1.3925x over previous
import jax
import jax.numpy as jnp
from jax import lax
from jax.experimental import pallas as pl
from jax.experimental.pallas import tpu as pltpu, tpu_sc as plsc

_N = 50000
_L = 20
_D = 50
_F = _L * _D
_BS = 500
_NB = _N // _BS
_CH = 2048
_NFULL = _N // _CH
_LAST = _N - _CH
_LC = 512
_NLFULL = _N // _LC
_LLAST = _N - _LC
_NC = 2
_NS = 16
_NW = _NC * _NS


def _body(xT, idx, lengths, outT, outlen,
          strip_v, cidx_v, outb_v, lidx_v, lout_v, sem):
    wid = lax.axis_index("s") * _NC + lax.axis_index("c")

    def len_body(i, carry):
        t = wid + i * _NW
        base = jnp.where(t < _NLFULL, t * _LC, _LLAST)
        pltpu.sync_copy(idx.at[pl.ds(base, _LC)], lidx_v)
        pltpu.async_copy(lengths.at[lidx_v], lout_v, sem).wait()
        pltpu.sync_copy(lout_v, outlen.at[pl.ds(base, _LC)])
        return carry

    lax.fori_loop(0, (_NLFULL + 1 - wid + _NW - 1) // _NW, len_body, 0)

    def row_body(i, carry):
        r = wid + i * _NW
        pltpu.sync_copy(xT.at[pl.ds(r, 1), :], strip_v)
        for c in range(_NFULL + 1):
            j0 = c * _CH if c < _NFULL else _LAST
            pltpu.sync_copy(idx.at[pl.ds(j0, _CH)], cidx_v)

            def gather_body(k, carry2):
                col = cidx_v[pl.ds(k * 16, 16)]
                vals = plsc.load_gather(strip_v, [jnp.zeros((16,), jnp.int32),
                                                  col])
                outb_v[0, pl.ds(j0 + k * 16, 16)] = vals
                return carry2

            lax.fori_loop(0, _CH // 16, gather_body, 0)
        pltpu.sync_copy(outb_v, outT.at[pl.ds(r, 1), :])
        return carry

    lax.fori_loop(0, (_F - wid + _NW - 1) // _NW, row_body, 0)


def kernel(x, lengths, promptList, batchSize):
    xT = x.reshape(_N, _F).T
    idx = promptList.astype(jnp.int32)
    mesh = plsc.VectorSubcoreMesh(core_axis_name="c", subcore_axis_name="s")
    outT, outlen = pl.kernel(
        _body,
        out_type=(
            jax.ShapeDtypeStruct((_F, _N), jnp.float32),
            jax.ShapeDtypeStruct((_N,), jnp.int32),
        ),
        mesh=mesh,
        scratch_types=[
            pltpu.VMEM((1, _N), jnp.float32),
            pltpu.VMEM((_CH,), jnp.int32),
            pltpu.VMEM((1, _N), jnp.float32),
            pltpu.VMEM((_LC,), jnp.int32),
            pltpu.VMEM((_LC,), jnp.int32),
            pltpu.SemaphoreType.DMA,
        ],
        compiler_params=pltpu.CompilerParams(needs_layout_passes=False),
    )(xT, idx, lengths)
    xList = outT.reshape(_L, _D, _NB, _BS).transpose(2, 3, 0, 1)
    lengthList = outlen.reshape(_NB, _BS) + jnp.asarray(
        batchSize - _BS, dtype=outlen.dtype)
    return (xList, lengthList)


# speedup vs baseline: 13.5949x; 2.7292x over previous
import jax
import jax.numpy as jnp
from jax import lax
from jax.experimental import pallas as pl
from jax.experimental.pallas import tpu as pltpu, tpu_sc as plsc

_N = 50000
_L = 20
_D = 50
_F = _L * _D
_BS = 500
_NB = _N // _BS
_CH = 2048
_NFULL = _N // _CH
_LAST = _N - _CH
_LC = 512
_NLFULL = _N // _LC
_LLAST = _N - _LC
_NC = 2
_NS = 16
_NW = _NC * _NS


def _body(xT, idx, lengths, outT, outlen,
          idx_sh, strip_v, cidx0_v, cidx1_v, outb_v, lidx_v, lout_v,
          sem, psem):
    cidx = (cidx0_v, cidx1_v)
    sid = lax.axis_index("s")
    wid = sid * _NC + lax.axis_index("c")

    @pl.when(sid == 0)
    def _():
        pltpu.sync_copy(idx, idx_sh)

    plsc.subcore_barrier()

    def len_body(i, carry):
        t = wid + i * _NW
        base = jnp.where(t < _NLFULL, t * _LC, _LLAST)
        pltpu.sync_copy(idx_sh.at[pl.ds(base, _LC)], lidx_v)
        pltpu.async_copy(lengths.at[lidx_v], lout_v, sem).wait()
        pltpu.sync_copy(lout_v, outlen.at[pl.ds(base, _LC)])
        return carry

    lax.fori_loop(0, (_NLFULL + 1 - wid + _NW - 1) // _NW, len_body, 0)

    zeros = jnp.zeros((16,), jnp.int32)

    def row_body(i, carry):
        r = wid + i * _NW
        pltpu.sync_copy(xT.at[pl.ds(r, 1), :], strip_v)
        cp = pltpu.async_copy(idx_sh.at[pl.ds(0, _CH)], cidx[0], psem)
        for c in range(_NFULL + 1):
            j0 = c * _CH if c < _NFULL else _LAST
            cp.wait()
            if c < _NFULL:
                nj0 = (c + 1) * _CH if c + 1 < _NFULL else _LAST
                cp = pltpu.async_copy(idx_sh.at[pl.ds(nj0, _CH)],
                                      cidx[(c + 1) % 2], psem)
            buf = cidx[c % 2]

            @plsc.parallel_loop(0, _CH, step=16, unroll=8)
            def gather(j):
                col = buf[pl.ds(j, 16)]
                vals = plsc.load_gather(strip_v, [zeros, col])
                outb_v[0, pl.ds(j0 + j, 16)] = vals

        pltpu.sync_copy(outb_v, outT.at[pl.ds(r, 1), :])
        return carry

    lax.fori_loop(0, (_F - wid + _NW - 1) // _NW, row_body, 0)


def kernel(x, lengths, promptList, batchSize):
    xT = x.reshape(_N, _F).T
    idx = promptList.astype(jnp.int32)
    mesh = plsc.VectorSubcoreMesh(core_axis_name="c", subcore_axis_name="s")
    outT, outlen = pl.kernel(
        _body,
        out_type=(
            jax.ShapeDtypeStruct((_F, _N), jnp.float32),
            jax.ShapeDtypeStruct((_N,), jnp.int32),
        ),
        mesh=mesh,
        scratch_types=[
            pltpu.VMEM_SHARED((_N,), jnp.int32),
            pltpu.VMEM((1, _N), jnp.float32),
            pltpu.VMEM((_CH,), jnp.int32),
            pltpu.VMEM((_CH,), jnp.int32),
            pltpu.VMEM((1, _N), jnp.float32),
            pltpu.VMEM((_LC,), jnp.int32),
            pltpu.VMEM((_LC,), jnp.int32),
            pltpu.SemaphoreType.DMA,
            pltpu.SemaphoreType.DMA,
        ],
        compiler_params=pltpu.CompilerParams(needs_layout_passes=False),
    )(xT, idx, lengths)
    xList = outT.reshape(_L, _D, _NB, _BS).transpose(2, 3, 0, 1)
    lengthList = outlen.reshape(_NB, _BS) + jnp.asarray(
        batchSize - _BS, dtype=outlen.dtype)
    return (xList, lengthList)


# speedup vs baseline: 14.3881x; 1.0583x over previous
import jax
import jax.numpy as jnp
from jax import lax
from jax.experimental import pallas as pl
from jax.experimental.pallas import tpu as pltpu, tpu_sc as plsc

_N = 50000
_L = 20
_D = 50
_F = _L * _D
_BS = 500
_NB = _N // _BS
_CH = 4096
_NFULL = _N // _CH
_LAST = _N - _CH
_LC = 512
_NLFULL = _N // _LC
_LLAST = _N - _LC
_NC = 2
_NS = 16
_NW = _NC * _NS


def _body(xT, idx, lengths, outT, outlen,
          idx_sh, strip_v, cidx0_v, cidx1_v, outb_v, lidx_v, lout_v,
          sem, psem, ssem, wsem):
    cidx = (cidx0_v, cidx1_v)
    sid = lax.axis_index("s")
    wid = sid * _NC + lax.axis_index("c")

    @pl.when(sid == 0)
    def _():
        pltpu.sync_copy(idx, idx_sh)

    plsc.subcore_barrier()

    def len_body(i, carry):
        t = wid + i * _NW
        base = jnp.where(t < _NLFULL, t * _LC, _LLAST)
        pltpu.sync_copy(idx_sh.at[pl.ds(base, _LC)], lidx_v)
        pltpu.async_copy(lengths.at[lidx_v], lout_v, sem).wait()
        pltpu.sync_copy(lout_v, outlen.at[pl.ds(base, _LC)])
        return carry

    lax.fori_loop(0, (_NLFULL + 1 - wid + _NW - 1) // _NW, len_body, 0)

    zeros = jnp.zeros((16,), jnp.int32)

    def row_body(i, carry):
        r = wid + i * _NW
        scp = pltpu.async_copy(xT.at[pl.ds(r, 1), :], strip_v, ssem)
        cp = pltpu.async_copy(idx_sh.at[pl.ds(0, _CH)], cidx[0], psem)

        @pl.when(i > 0)
        def _():
            pltpu.make_async_copy(outb_v, outT.at[pl.ds(r, 1), :],
                                  wsem).wait()

        scp.wait()
        for c in range(_NFULL + 1):
            j0 = c * _CH if c < _NFULL else _LAST
            cp.wait()
            if c < _NFULL:
                nj0 = (c + 1) * _CH if c + 1 < _NFULL else _LAST
                cp = pltpu.async_copy(idx_sh.at[pl.ds(nj0, _CH)],
                                      cidx[(c + 1) % 2], psem)
            buf = cidx[c % 2]

            @plsc.parallel_loop(0, _CH, step=16, unroll=8)
            def gather(j):
                col = buf[pl.ds(j, 16)]
                vals = plsc.load_gather(strip_v, [zeros, col])
                outb_v[0, pl.ds(j0 + j, 16)] = vals

        pltpu.async_copy(outb_v, outT.at[pl.ds(r, 1), :], wsem)
        return carry

    lax.fori_loop(0, (_F - wid + _NW - 1) // _NW, row_body, 0)
    pltpu.make_async_copy(outb_v, outT.at[pl.ds(wid, 1), :], wsem).wait()


def kernel(x, lengths, promptList, batchSize):
    xT = x.reshape(_N, _F).T
    idx = promptList.astype(jnp.int32)
    mesh = plsc.VectorSubcoreMesh(core_axis_name="c", subcore_axis_name="s")
    outT, outlen = pl.kernel(
        _body,
        out_type=(
            jax.ShapeDtypeStruct((_F, _N), jnp.float32),
            jax.ShapeDtypeStruct((_N,), jnp.int32),
        ),
        mesh=mesh,
        scratch_types=[
            pltpu.VMEM_SHARED((_N,), jnp.int32),
            pltpu.VMEM((1, _N), jnp.float32),
            pltpu.VMEM((_CH,), jnp.int32),
            pltpu.VMEM((_CH,), jnp.int32),
            pltpu.VMEM((1, _N), jnp.float32),
            pltpu.VMEM((_LC,), jnp.int32),
            pltpu.VMEM((_LC,), jnp.int32),
            pltpu.SemaphoreType.DMA,
            pltpu.SemaphoreType.DMA,
            pltpu.SemaphoreType.DMA,
            pltpu.SemaphoreType.DMA,
        ],
        compiler_params=pltpu.CompilerParams(needs_layout_passes=False),
    )(xT, idx, lengths)
    xList = outT.reshape(_L, _D, _NB, _BS).transpose(2, 3, 0, 1)
    lengthList = outlen.reshape(_NB, _BS) + jnp.asarray(
        batchSize - _BS, dtype=outlen.dtype)
    return (xList, lengthList)


# speedup vs baseline: 17.0680x; 1.1863x over previous
import jax
import jax.numpy as jnp
from jax import lax
from jax.experimental import pallas as pl
from jax.experimental.pallas import tpu as pltpu, tpu_sc as plsc

_N = 50000
_NP = 50048
_L = 20
_D = 50
_F = _L * _D
_BS = 500
_NB = _N // _BS
_CH = 4096
_NFULL = _N // _CH
_TL0 = _NFULL * _CH
_TW = _NP - _TL0
_TREAL = _N - _TL0
_NCH = _NFULL + 1
_LC = 512
_NLFULL = _N // _LC
_LLAST = _N - _LC
_NC = 2
_NS = 16
_NW = _NC * _NS


def _chw(c):
    return (c * _CH, _CH) if c < _NFULL else (_TL0, _TW)


def _body(xT, idx, lengths, outT, outlen,
          idx_sh, stripA, stripB, cidx0, cidx1, outc0, outc1,
          lidx_v, lout_v, sem, psem, ssemA, ssemB, wsem0, wsem1):
    cidx = (cidx0, cidx1)
    outc = (outc0, outc1)
    wsem = (wsem0, wsem1)
    sid = lax.axis_index("s")
    wid = sid * _NC + lax.axis_index("c")

    @pl.when(sid == 0)
    def _():
        pltpu.sync_copy(idx, idx_sh)

    plsc.subcore_barrier()

    def len_body(i, carry):
        t = wid + i * _NW
        base = jnp.where(t < _NLFULL, t * _LC, _LLAST)
        pltpu.sync_copy(idx_sh.at[pl.ds(base, _LC)], lidx_v)
        pltpu.async_copy(lengths.at[lidx_v], lout_v, sem).wait()
        pltpu.sync_copy(lout_v, outlen.at[pl.ds(base, _LC)])
        return carry

    lax.fori_loop(0, (_NLFULL + 1 - wid + _NW - 1) // _NW, len_body, 0)

    ntrips = (_F - wid + _NW - 1) // _NW
    npairs = ntrips // 2
    iota = lax.iota(jnp.int32, 16)

    def drain(c, r):
        j0, w = _chw(c)
        pltpu.make_async_copy(
            outc[c % 2].at[:, pl.ds(0, w)],
            outT.at[pl.ds(r, 1), pl.ds(j0, w)], wsem[c % 2]).wait()

    def gather_row(r, strip, has_prev):
        cp = pltpu.async_copy(idx_sh.at[pl.ds(0, _CH)], cidx[0], psem)
        for c in range(_NCH):
            j0, w = _chw(c)
            cp.wait()
            if c + 1 < _NCH:
                nj0, nw = _chw(c + 1)
                cp = pltpu.async_copy(idx_sh.at[pl.ds(nj0, nw)],
                                      cidx[(c + 1) % 2].at[pl.ds(0, nw)],
                                      psem)
            buf = cidx[c % 2]
            ob = outc[c % 2]
            if c >= 2:
                drain(c - 2, r)
            else:
                @pl.when(has_prev)
                def _():
                    drain(c + _NCH - 2, r)

            if c < _NFULL:
                @plsc.parallel_loop(0, _CH, step=16, unroll=8)
                def gather(j):
                    col = buf[pl.ds(j, 16)]
                    vals = plsc.load_gather(strip, [jnp.zeros((16,),
                                                             jnp.int32), col])
                    ob[0, pl.ds(j, 16)] = vals
            else:
                @plsc.parallel_loop(0, _TW, step=16, unroll=8)
                def gather_tail(j):
                    col = buf[pl.ds(j, 16)]
                    m = (iota + j) < _TREAL
                    vals = plsc.load_gather(strip, [jnp.zeros((16,),
                                                             jnp.int32), col],
                                            mask=m)
                    ob[0, pl.ds(j, 16)] = vals

            pltpu.async_copy(ob.at[:, pl.ds(0, w)],
                             outT.at[pl.ds(r, 1), pl.ds(j0, w)],
                             wsem[c % 2])

    scpA = pltpu.async_copy(xT.at[pl.ds(wid, 1), :], stripA, ssemA)

    def pair_body(i, carry):
        rA = wid + (2 * i) * _NW
        rB = rA + _NW
        pltpu.make_async_copy(xT.at[pl.ds(rA, 1), :], stripA, ssemA).wait()
        pltpu.async_copy(xT.at[pl.ds(rB, 1), :], stripB, ssemB)
        gather_row(rA, stripA, i > 0)

        pltpu.make_async_copy(xT.at[pl.ds(rB, 1), :], stripB, ssemB).wait()
        rC = rB + _NW

        @pl.when(2 * i + 2 < ntrips)
        def _():
            pltpu.async_copy(xT.at[pl.ds(rC, 1), :], stripA, ssemA)

        gather_row(rB, stripB, True)
        return carry

    lax.fori_loop(0, npairs, pair_body, 0)

    @pl.when(ntrips % 2 == 1)
    def _():
        r = wid + (ntrips - 1) * _NW
        pltpu.make_async_copy(xT.at[pl.ds(r, 1), :], stripA, ssemA).wait()
        gather_row(r, stripA, npairs > 0)

    drain(_NCH - 2, wid)
    drain(_NCH - 1, wid)


def kernel(x, lengths, promptList, batchSize):
    xT = x.reshape(_N, _F).T
    idx = jnp.pad(promptList.astype(jnp.int32), (0, _NP - _N))
    mesh = plsc.VectorSubcoreMesh(core_axis_name="c", subcore_axis_name="s")
    outT, outlen = pl.kernel(
        _body,
        out_type=(
            jax.ShapeDtypeStruct((_F, _NP), jnp.float32),
            jax.ShapeDtypeStruct((_N,), jnp.int32),
        ),
        mesh=mesh,
        scratch_types=[
            pltpu.VMEM_SHARED((_NP,), jnp.int32),
            pltpu.VMEM((1, _N), jnp.float32),
            pltpu.VMEM((1, _N), jnp.float32),
            pltpu.VMEM((_CH,), jnp.int32),
            pltpu.VMEM((_CH,), jnp.int32),
            pltpu.VMEM((1, _CH), jnp.float32),
            pltpu.VMEM((1, _CH), jnp.float32),
            pltpu.VMEM((_LC,), jnp.int32),
            pltpu.VMEM((_LC,), jnp.int32),
            pltpu.SemaphoreType.DMA,
            pltpu.SemaphoreType.DMA,
            pltpu.SemaphoreType.DMA,
            pltpu.SemaphoreType.DMA,
            pltpu.SemaphoreType.DMA,
            pltpu.SemaphoreType.DMA,
        ],
        compiler_params=pltpu.CompilerParams(needs_layout_passes=False),
    )(xT, idx, lengths)
    xList = outT[:, :_N].reshape(_L, _D, _NB, _BS).transpose(2, 3, 0, 1)
    lengthList = outlen.reshape(_NB, _BS) + jnp.asarray(
        batchSize - _BS, dtype=outlen.dtype)
    return (xList, lengthList)
